# baseline (speedup 1.0000x reference)
import jax
import jax.numpy as jnp
from jax.experimental import pallas as pl
from jax.experimental.pallas import tpu as pltpu

_PARAM = 0.1
_BIG = 1e10
_B = 4
_N = 256
_P = 50176
_TP = 6272
_STEPS = _P // _TP


def _body(x_ref, t_ref, out_ref, ysum_ref, cnt_ref, minx_ref):
    step = pl.program_id(0)

    @pl.when(step == 0)
    def _init():
        ysum_ref[...] = jnp.zeros_like(ysum_ref)
        cnt_ref[...] = jnp.zeros_like(cnt_ref)
        minx_ref[...] = jnp.full_like(minx_ref, _BIG)

    x = x_ref[...]
    t = t_ref[...]
    mask = t >= 0.1
    for b in range(_B):
        d2 = (x[b, :, None] - t[b, None, :]) ** 2
        min_y = jnp.min(d2, axis=0)
        contrib = jnp.where(mask[b], min_y, 0.0).reshape(_TP // 128, 128)
        ysum_ref[b, :] += jnp.sum(contrib, axis=0)
        cnt_ref[b, :] += jnp.sum(
            mask[b].astype(jnp.float32).reshape(_TP // 128, 128), axis=0)
        min_x = jnp.min(jnp.where(mask[b][None, :], d2, _BIG), axis=1)
        minx_ref[b, :] = jnp.minimum(minx_ref[b, :], min_x)

    @pl.when(step == _STEPS - 1)
    def _finish():
        cham_x = jnp.mean(minx_ref[...], axis=1)
        cham_y = jnp.sum(ysum_ref[...], axis=1) / jnp.sum(cnt_ref[...], axis=1)
        loss = _PARAM * jnp.mean(cham_x + cham_y)
        out_ref[...] = jnp.full((1, 128), loss, jnp.float32)


def kernel(bins, target_depth_maps):
    x = jnp.reshape(bins, (_B, _N))
    t = jnp.reshape(target_depth_maps, (_B, _P))
    out = pl.pallas_call(
        _body,
        grid=(_STEPS,),
        in_specs=[
            pl.BlockSpec((_B, _N), lambda i: (0, 0)),
            pl.BlockSpec((_B, _TP), lambda i: (0, i)),
        ],
        out_specs=pl.BlockSpec((1, 128), lambda i: (0, 0)),
        out_shape=jax.ShapeDtypeStruct((1, 128), jnp.float32),
        scratch_shapes=[
            pltpu.VMEM((_B, 128), jnp.float32),
            pltpu.VMEM((_B, 128), jnp.float32),
            pltpu.VMEM((_B, _N), jnp.float32),
        ],
    )(x, t)
    return out[0, 0]


# speedup vs baseline: 2.3827x; 2.3827x over previous
import functools

import jax
import jax.numpy as jnp
from jax import lax
from jax.experimental import pallas as pl
from jax.experimental.pallas import tpu as pltpu
from jax.experimental.pallas import tpu_sc as plsc

_PARAM = 0.1
_BIG = 1e10
_B = 4
_N = 256
_P = 50176
_NW = 32
_CHUNK = _P // _NW
_JV = _CHUNK // 16
_GAPS = 264
_GROW = _GAPS * 16


def _sc_body(x_hbm, t_hbm, ysum_hbm, cnt_hbm, lo_hbm, hi_hbm, sortx_hbm,
             binv, sortv, myranks, allranks, shranks, tv, lo2d, hi2d, stage):
    c = lax.axis_index("c")
    s = lax.axis_index("s")
    wid = s * 2 + c
    lane = lax.iota(jnp.int32, 16)

    pltpu.sync_copy(x_hbm, binv)

    for v in range(4):
        base = s * 64 + v * 16
        gi = base + lane
        bb = base // 256
        vals = binv[pl.ds(base, 16)]

        def rank_step(m, r):
            src = bb * 256 + m
            splat = plsc.load_gather(binv, [jnp.zeros((16,), jnp.int32) + src])
            less = (splat < vals) | ((splat == vals) & (src < gi))
            return r + less.astype(jnp.int32)

        rank = lax.fori_loop(0, 256, rank_step, jnp.zeros((16,), jnp.int32))
        myranks[pl.ds(v * 16, 16)] = rank

    pltpu.sync_copy(myranks, shranks.at[pl.ds(s * 64, 64)])
    plsc.subcore_barrier()
    pltpu.sync_copy(shranks, allranks)

    for j in range(64):
        r = allranks[pl.ds(j * 16, 16)]
        v = binv[pl.ds(j * 16, 16)]
        dest = (j // 16) * 256 + r
        plsc.store_scatter(sortv, [dest], v)

    for b in range(_B):
        def init_row(r, carry):
            lo2d[pl.ds(r * 16, 16)] = jnp.full((16,), -_BIG, jnp.float32)
            hi2d[pl.ds(r * 16, 16)] = jnp.full((16,), _BIG, jnp.float32)
            return carry

        lax.fori_loop(0, _GAPS, init_row, 0)

        tbase = b * _P + wid * _CHUNK
        pltpu.sync_copy(t_hbm.at[pl.ds(tbase, _CHUNK)], tv)

        def target_step(j, carry):
            acc_y, acc_c = carry
            t = tv[pl.ds(j * 16, 16)]
            valid = t >= 0.1
            i = jnp.zeros((16,), jnp.int32)
            for k in (128, 64, 32, 16, 8, 4, 2, 1):
                g = plsc.load_gather(sortv, [b * 256 + i + (k - 1)])
                i = i + jnp.where(g < t, k, 0)
            lv = plsc.load_gather(sortv, [b * 256 + jnp.maximum(i - 1, 0)])
            hv = plsc.load_gather(sortv, [b * 256 + jnp.minimum(i, 255)])
            dl = t - lv
            dh = hv - t
            dlo = jnp.where(i > 0, dl * dl, _BIG)
            dhi = jnp.where(i < 256, dh * dh, _BIG)
            d = jnp.minimum(dlo, dhi)
            acc_y = acc_y + jnp.where(valid, d, 0.0)
            acc_c = acc_c + jnp.where(valid, 1.0, 0.0)
            fi = i * 16 + lane
            tlo = jnp.where(valid, t, -_BIG)
            thi = jnp.where(valid, t, _BIG)
            cur = plsc.load_gather(lo2d, [fi])
            plsc.store_scatter(lo2d, [fi], jnp.maximum(cur, tlo))
            cur = plsc.load_gather(hi2d, [fi])
            plsc.store_scatter(hi2d, [fi], jnp.minimum(cur, thi))
            return acc_y, acc_c

        zero = jnp.zeros((16,), jnp.float32)
        acc_y, acc_c = lax.fori_loop(0, _JV, target_step, (zero, zero))

        slot = b * _NW + wid
        stage[...] = acc_y
        pltpu.sync_copy(stage, ysum_hbm.at[pl.ds(slot * 16, 16)])
        stage[...] = acc_c
        pltpu.sync_copy(stage, cnt_hbm.at[pl.ds(slot * 16, 16)])
        pltpu.sync_copy(lo2d, lo_hbm.at[pl.ds(slot * _GROW, _GROW)])
        pltpu.sync_copy(hi2d, hi_hbm.at[pl.ds(slot * _GROW, _GROW)])

    @pl.when(wid == 0)
    def _():
        pltpu.sync_copy(sortv, sortx_hbm)


def _sc_call(x_flat, t_flat):
    mesh = plsc.VectorSubcoreMesh(core_axis_name="c", subcore_axis_name="s")
    f32 = jnp.float32
    out_type = (
        jax.ShapeDtypeStruct((_B * _NW * 16,), f32),
        jax.ShapeDtypeStruct((_B * _NW * 16,), f32),
        jax.ShapeDtypeStruct((_B * _NW * _GROW,), f32),
        jax.ShapeDtypeStruct((_B * _NW * _GROW,), f32),
        jax.ShapeDtypeStruct((_B * _N,), f32),
    )
    scratch = [
        pltpu.VMEM((_B * _N,), f32),
        pltpu.VMEM((_B * _N,), f32),
        pltpu.VMEM((64,), jnp.int32),
        pltpu.VMEM((_B * _N,), jnp.int32),
        pltpu.VMEM_SHARED((_B * _N,), jnp.int32),
        pltpu.VMEM((_CHUNK,), f32),
        pltpu.VMEM((_GROW,), f32),
        pltpu.VMEM((_GROW,), f32),
        pltpu.VMEM((16,), f32),
    ]
    kern = functools.partial(
        pl.kernel, mesh=mesh, out_type=out_type, scratch_types=scratch,
        compiler_params=pltpu.CompilerParams(needs_layout_passes=False),
    )(_sc_body)
    return kern(x_flat, t_flat)


def _tc_epilogue(ys_ref, cn_ref, lo_ref, hi_ref, sx_ref, out_ref):
    ys = ys_ref[...]
    cn = cn_ref[...]
    lo = lo_ref[...].reshape(_B, _NW, _GAPS, 16)
    hi = hi_ref[...].reshape(_B, _NW, _GAPS, 16)
    sx = sx_ref[...]
    lor = jnp.max(jnp.max(lo, axis=3), axis=1)
    hir = jnp.min(jnp.min(hi, axis=3), axis=1)
    for sh in (1, 2, 4, 8, 16, 32, 64, 128, 256):
        pad = jnp.full((_B, sh), -_BIG, jnp.float32)
        lor = jnp.maximum(lor, jnp.concatenate([pad, lor[:, :_GAPS - sh]], axis=1))
        pad = jnp.full((_B, sh), _BIG, jnp.float32)
        hir = jnp.minimum(hir, jnp.concatenate([hir[:, sh:], pad], axis=1))
    lo_bin = lor[:, :_N]
    hi_bin = hir[:, 1:_N + 1]
    minx = jnp.minimum((sx - lo_bin) ** 2, (hi_bin - sx) ** 2)
    cham_x = jnp.mean(minx, axis=1)
    cham_y = jnp.sum(ys, axis=1) / jnp.sum(cn, axis=1)
    loss = _PARAM * jnp.mean(cham_x + cham_y)
    out_ref[...] = jnp.full((1, 128), loss, jnp.float32)


def kernel(bins, target_depth_maps):
    x_flat = jnp.reshape(bins, (_B * _N,))
    t_flat = jnp.reshape(target_depth_maps, (_B * _P,))
    ysum, cnt, loseed, hiseed, sortx = _sc_call(x_flat, t_flat)
    out = pl.pallas_call(
        _tc_epilogue,
        out_shape=jax.ShapeDtypeStruct((1, 128), jnp.float32),
    )(
        ysum.reshape(_B, _NW * 16),
        cnt.reshape(_B, _NW * 16),
        loseed.reshape(_B, _NW * _GROW),
        hiseed.reshape(_B, _NW * _GROW),
        sortx.reshape(_B, _N),
    )
    return out[0, 0]


# speedup vs baseline: 2.4284x; 1.0192x over previous
import functools

import jax
import jax.numpy as jnp
from jax import lax
from jax.experimental import pallas as pl
from jax.experimental.pallas import tpu as pltpu
from jax.experimental.pallas import tpu_sc as plsc

_PARAM = 0.1
_BIG = 1e10
_B = 4
_N = 256
_P = 50176
_NW = 32
_CHUNK = _P // _NW
_JV = _CHUNK // 16
_GAPS = 264
_GROW = _GAPS * 16


def _sc_body(x_hbm, t_hbm, ysum_hbm, cnt_hbm, lo_hbm, hi_hbm, sortx_hbm,
             binv, sortv, myranks, allranks, shranks, tv, lo2d, hi2d, stage):
    c = lax.axis_index("c")
    s = lax.axis_index("s")
    wid = s * 2 + c
    lane = lax.iota(jnp.int32, 16)

    pltpu.sync_copy(x_hbm, binv)

    for v in range(4):
        base = s * 64 + v * 16
        gi = base + lane
        bb = base // 256
        vals = binv[pl.ds(base, 16)]

        def rank_step(m, r):
            src = bb * 256 + m
            splat = plsc.load_gather(binv, [jnp.zeros((16,), jnp.int32) + src])
            less = (splat < vals) | ((splat == vals) & (src < gi))
            return r + less.astype(jnp.int32)

        rank = lax.fori_loop(0, 256, rank_step, jnp.zeros((16,), jnp.int32),
                             unroll=8)
        myranks[pl.ds(v * 16, 16)] = rank

    pltpu.sync_copy(myranks, shranks.at[pl.ds(s * 64, 64)])
    plsc.subcore_barrier()
    pltpu.sync_copy(shranks, allranks)

    for j in range(64):
        r = allranks[pl.ds(j * 16, 16)]
        v = binv[pl.ds(j * 16, 16)]
        dest = (j // 16) * 256 + r
        plsc.store_scatter(sortv, [dest], v)

    for b in range(_B):
        def init_row(r, carry):
            lo2d[pl.ds(r * 16, 16)] = jnp.full((16,), -_BIG, jnp.float32)
            hi2d[pl.ds(r * 16, 16)] = jnp.full((16,), _BIG, jnp.float32)
            return carry

        lax.fori_loop(0, _GAPS, init_row, 0, unroll=8)

        tbase = b * _P + wid * _CHUNK
        pltpu.sync_copy(t_hbm.at[pl.ds(tbase, _CHUNK)], tv)

        def target_step(j, carry):
            acc_y, acc_c = carry
            t = tv[pl.ds(j * 16, 16)]
            valid = t >= 0.1
            i = jnp.zeros((16,), jnp.int32)
            for k in (128, 64, 32, 16, 8, 4, 2, 1):
                g = plsc.load_gather(sortv, [b * 256 + i + (k - 1)])
                i = i + jnp.where(g < t, k, 0)
            lv = plsc.load_gather(sortv, [b * 256 + jnp.maximum(i - 1, 0)])
            hv = plsc.load_gather(sortv, [b * 256 + jnp.minimum(i, 255)])
            dl = t - lv
            dh = hv - t
            dlo = jnp.where(i > 0, dl * dl, _BIG)
            dhi = jnp.where(i < 256, dh * dh, _BIG)
            d = jnp.minimum(dlo, dhi)
            acc_y = acc_y + jnp.where(valid, d, 0.0)
            acc_c = acc_c + jnp.where(valid, 1.0, 0.0)
            fi = i * 16 + lane
            tlo = jnp.where(valid, t, -_BIG)
            thi = jnp.where(valid, t, _BIG)
            cur = plsc.load_gather(lo2d, [fi])
            plsc.store_scatter(lo2d, [fi], jnp.maximum(cur, tlo))
            cur = plsc.load_gather(hi2d, [fi])
            plsc.store_scatter(hi2d, [fi], jnp.minimum(cur, thi))
            return acc_y, acc_c

        zero = jnp.zeros((16,), jnp.float32)
        acc_y, acc_c = lax.fori_loop(0, _JV, target_step, (zero, zero),
                                     unroll=7)

        slot = b * _NW + wid
        stage[...] = acc_y
        pltpu.sync_copy(stage, ysum_hbm.at[pl.ds(slot * 16, 16)])
        stage[...] = acc_c
        pltpu.sync_copy(stage, cnt_hbm.at[pl.ds(slot * 16, 16)])
        pltpu.sync_copy(lo2d, lo_hbm.at[pl.ds(slot * _GROW, _GROW)])
        pltpu.sync_copy(hi2d, hi_hbm.at[pl.ds(slot * _GROW, _GROW)])

    @pl.when(wid == 0)
    def _():
        pltpu.sync_copy(sortv, sortx_hbm)


def _sc_call(x_flat, t_flat):
    mesh = plsc.VectorSubcoreMesh(core_axis_name="c", subcore_axis_name="s")
    f32 = jnp.float32
    out_type = (
        jax.ShapeDtypeStruct((_B * _NW * 16,), f32),
        jax.ShapeDtypeStruct((_B * _NW * 16,), f32),
        jax.ShapeDtypeStruct((_B * _NW * _GROW,), f32),
        jax.ShapeDtypeStruct((_B * _NW * _GROW,), f32),
        jax.ShapeDtypeStruct((_B * _N,), f32),
    )
    scratch = [
        pltpu.VMEM((_B * _N,), f32),
        pltpu.VMEM((_B * _N,), f32),
        pltpu.VMEM((64,), jnp.int32),
        pltpu.VMEM((_B * _N,), jnp.int32),
        pltpu.VMEM_SHARED((_B * _N,), jnp.int32),
        pltpu.VMEM((_CHUNK,), f32),
        pltpu.VMEM((_GROW,), f32),
        pltpu.VMEM((_GROW,), f32),
        pltpu.VMEM((16,), f32),
    ]
    kern = functools.partial(
        pl.kernel, mesh=mesh, out_type=out_type, scratch_types=scratch,
        compiler_params=pltpu.CompilerParams(needs_layout_passes=False),
    )(_sc_body)
    return kern(x_flat, t_flat)


def _tc_epilogue(ys_ref, cn_ref, lo_ref, hi_ref, sx_ref, out_ref):
    ys = ys_ref[...]
    cn = cn_ref[...]
    lo = lo_ref[...].reshape(_B, _NW, _GAPS, 16)
    hi = hi_ref[...].reshape(_B, _NW, _GAPS, 16)
    sx = sx_ref[...]
    lor = jnp.max(jnp.max(lo, axis=3), axis=1)
    hir = jnp.min(jnp.min(hi, axis=3), axis=1)
    for sh in (1, 2, 4, 8, 16, 32, 64, 128, 256):
        pad = jnp.full((_B, sh), -_BIG, jnp.float32)
        lor = jnp.maximum(lor, jnp.concatenate([pad, lor[:, :_GAPS - sh]], axis=1))
        pad = jnp.full((_B, sh), _BIG, jnp.float32)
        hir = jnp.minimum(hir, jnp.concatenate([hir[:, sh:], pad], axis=1))
    lo_bin = lor[:, :_N]
    hi_bin = hir[:, 1:_N + 1]
    minx = jnp.minimum((sx - lo_bin) ** 2, (hi_bin - sx) ** 2)
    cham_x = jnp.mean(minx, axis=1)
    cham_y = jnp.sum(ys, axis=1) / jnp.sum(cn, axis=1)
    loss = _PARAM * jnp.mean(cham_x + cham_y)
    out_ref[...] = jnp.full((1, 128), loss, jnp.float32)


def kernel(bins, target_depth_maps):
    x_flat = jnp.reshape(bins, (_B * _N,))
    t_flat = jnp.reshape(target_depth_maps, (_B * _P,))
    ysum, cnt, loseed, hiseed, sortx = _sc_call(x_flat, t_flat)
    out = pl.pallas_call(
        _tc_epilogue,
        out_shape=jax.ShapeDtypeStruct((1, 128), jnp.float32),
    )(
        ysum.reshape(_B, _NW * 16),
        cnt.reshape(_B, _NW * 16),
        loseed.reshape(_B, _NW * _GROW),
        hiseed.reshape(_B, _NW * _GROW),
        sortx.reshape(_B, _N),
    )
    return out[0, 0]


# speedup vs baseline: 2.5986x; 1.0701x over previous
import functools

import jax
import jax.numpy as jnp
from jax import lax
from jax.experimental import pallas as pl
from jax.experimental.pallas import tpu as pltpu
from jax.experimental.pallas import tpu_sc as plsc

_PARAM = 0.1
_BIG = 1e10
_B = 4
_N = 256
_P = 50176
_NW = 32
_CHUNK = _P // _NW
_JV = _CHUNK // 16
_GAPS = 264
_GROW = _GAPS * 16


def _sc_body(x_hbm, t_hbm, yc_hbm, lo_hbm, hi_hbm, sortx_hbm,
             binv, sortv, myranks, allranks, shranks, tv, lo2d, hi2d, yc,
             sems):
    c = lax.axis_index("c")
    s = lax.axis_index("s")
    wid = s * 2 + c
    lane = lax.iota(jnp.int32, 16)

    tcopies = []
    for b in range(_B):
        tbase = b * _P + wid * _CHUNK
        tcopies.append(pltpu.async_copy(
            t_hbm.at[pl.ds(tbase, _CHUNK)], tv.at[pl.ds(b * _CHUNK, _CHUNK)],
            sems.at[b]))

    pltpu.sync_copy(x_hbm, binv)

    for v in range(4):
        base = s * 64 + v * 16
        gi = base + lane
        bb = base // 256
        vals = binv[pl.ds(base, 16)]

        def rank_step(m, r):
            src = bb * 256 + m
            splat = plsc.load_gather(binv, [jnp.zeros((16,), jnp.int32) + src])
            less = (splat < vals) | ((splat == vals) & (src < gi))
            return r + less.astype(jnp.int32)

        rank = lax.fori_loop(0, 256, rank_step, jnp.zeros((16,), jnp.int32),
                             unroll=8)
        myranks[pl.ds(v * 16, 16)] = rank

    def init_row(r, carry):
        lo2d[pl.ds(r * 16, 16)] = jnp.full((16,), -_BIG, jnp.float32)
        hi2d[pl.ds(r * 16, 16)] = jnp.full((16,), _BIG, jnp.float32)
        return carry

    pltpu.sync_copy(myranks, shranks.at[pl.ds(s * 64, 64)])
    lax.fori_loop(0, _B * _GAPS, init_row, 0, unroll=8)
    plsc.subcore_barrier()
    pltpu.sync_copy(shranks, allranks)

    for j in range(64):
        r = allranks[pl.ds(j * 16, 16)]
        v = binv[pl.ds(j * 16, 16)]
        dest = (j // 16) * 256 + r
        plsc.store_scatter(sortv, [dest], v)

    @pl.when(wid == 0)
    def _():
        pltpu.sync_copy(sortv, sortx_hbm)

    for b in range(_B):
        tcopies[b].wait()

        def target_step(j, carry):
            acc_y, acc_c = carry
            t = tv[pl.ds(b * _CHUNK + j * 16, 16)]
            valid = t >= 0.1
            i = jnp.zeros((16,), jnp.int32)
            for k in (128, 64, 32, 16, 8, 4, 2, 1):
                g = plsc.load_gather(sortv, [b * 256 + i + (k - 1)])
                i = i + jnp.where(g < t, k, 0)
            lv = plsc.load_gather(sortv, [b * 256 + jnp.maximum(i - 1, 0)])
            hv = plsc.load_gather(sortv, [b * 256 + jnp.minimum(i, 255)])
            dl = t - lv
            dh = hv - t
            dlo = jnp.where(i > 0, dl * dl, _BIG)
            dhi = jnp.where(i < 256, dh * dh, _BIG)
            d = jnp.minimum(dlo, dhi)
            acc_y = acc_y + jnp.where(valid, d, 0.0)
            acc_c = acc_c + jnp.where(valid, 1.0, 0.0)
            fi = b * _GROW + i * 16 + lane
            tlo = jnp.where(valid, t, -_BIG)
            thi = jnp.where(valid, t, _BIG)
            cur = plsc.load_gather(lo2d, [fi])
            plsc.store_scatter(lo2d, [fi], jnp.maximum(cur, tlo))
            cur = plsc.load_gather(hi2d, [fi])
            plsc.store_scatter(hi2d, [fi], jnp.minimum(cur, thi))
            return acc_y, acc_c

        zero = jnp.zeros((16,), jnp.float32)
        acc_y, acc_c = lax.fori_loop(0, _JV, target_step, (zero, zero),
                                     unroll=7)
        yc[pl.ds(b * 32, 16)] = acc_y
        yc[pl.ds(b * 32 + 16, 16)] = acc_c

    pltpu.sync_copy(yc, yc_hbm.at[pl.ds(wid * 128, 128)])
    pltpu.sync_copy(lo2d, lo_hbm.at[pl.ds(wid * _B * _GROW, _B * _GROW)])
    pltpu.sync_copy(hi2d, hi_hbm.at[pl.ds(wid * _B * _GROW, _B * _GROW)])


def _sc_call(x_flat, t_flat):
    mesh = plsc.VectorSubcoreMesh(core_axis_name="c", subcore_axis_name="s")
    f32 = jnp.float32
    out_type = (
        jax.ShapeDtypeStruct((_NW * 128,), f32),
        jax.ShapeDtypeStruct((_NW * _B * _GROW,), f32),
        jax.ShapeDtypeStruct((_NW * _B * _GROW,), f32),
        jax.ShapeDtypeStruct((_B * _N,), f32),
    )
    scratch = [
        pltpu.VMEM((_B * _N,), f32),
        pltpu.VMEM((_B * _N,), f32),
        pltpu.VMEM((64,), jnp.int32),
        pltpu.VMEM((_B * _N,), jnp.int32),
        pltpu.VMEM_SHARED((_B * _N,), jnp.int32),
        pltpu.VMEM((_B * _CHUNK,), f32),
        pltpu.VMEM((_B * _GROW,), f32),
        pltpu.VMEM((_B * _GROW,), f32),
        pltpu.VMEM((128,), f32),
        pltpu.SemaphoreType.DMA((_B,)),
    ]
    kern = functools.partial(
        pl.kernel, mesh=mesh, out_type=out_type, scratch_types=scratch,
        compiler_params=pltpu.CompilerParams(needs_layout_passes=False),
    )(_sc_body)
    return kern(x_flat, t_flat)


def _tc_epilogue(yc_ref, lo_ref, hi_ref, sx_ref, out_ref):
    yc = yc_ref[...].reshape(_NW, _B, 2, 16)
    lo = lo_ref[...].reshape(_NW, _B, _GAPS, 16)
    hi = hi_ref[...].reshape(_NW, _B, _GAPS, 16)
    sx = sx_ref[...]
    lor = jnp.max(jnp.max(lo, axis=3), axis=0)
    hir = jnp.min(jnp.min(hi, axis=3), axis=0)
    for sh in (1, 2, 4, 8, 16, 32, 64, 128, 256):
        pad = jnp.full((_B, sh), -_BIG, jnp.float32)
        lor = jnp.maximum(lor, jnp.concatenate([pad, lor[:, :_GAPS - sh]], axis=1))
        pad = jnp.full((_B, sh), _BIG, jnp.float32)
        hir = jnp.minimum(hir, jnp.concatenate([hir[:, sh:], pad], axis=1))
    lo_bin = lor[:, :_N]
    hi_bin = hir[:, 1:_N + 1]
    minx = jnp.minimum((sx - lo_bin) ** 2, (hi_bin - sx) ** 2)
    cham_x = jnp.mean(minx, axis=1)
    ysums = jnp.sum(jnp.sum(yc[:, :, 0, :], axis=2), axis=0)
    cnts = jnp.sum(jnp.sum(yc[:, :, 1, :], axis=2), axis=0)
    cham_y = ysums / cnts
    loss = _PARAM * jnp.mean(cham_x + cham_y)
    out_ref[...] = jnp.full((1, 128), loss, jnp.float32)


def kernel(bins, target_depth_maps):
    x_flat = jnp.reshape(bins, (_B * _N,))
    t_flat = jnp.reshape(target_depth_maps, (_B * _P,))
    yc, loseed, hiseed, sortx = _sc_call(x_flat, t_flat)
    out = pl.pallas_call(
        _tc_epilogue,
        out_shape=jax.ShapeDtypeStruct((1, 128), jnp.float32),
    )(
        yc.reshape(_NW, _B * 2 * 16),
        loseed.reshape(_NW, _B * _GROW),
        hiseed.reshape(_NW, _B * _GROW),
        sortx.reshape(_B, _N),
    )
    return out[0, 0]
